# baseline (speedup 1.0000x reference)
import functools

import jax
import jax.numpy as jnp
from jax import lax
from jax.experimental import pallas as pl
from jax.experimental.pallas import tpu as pltpu
from jax.experimental.pallas import tpu_sc as plsc

BT = 128
BTE = BT + 8
NC = 2
NS = 16
NW = NC * NS
CH = 64


def _gate_body(x_ref, wg_ref, idx_ref, w_ref):
    logits = jnp.dot(x_ref[...], wg_ref[...], preferred_element_type=jnp.float32)
    m = jnp.max(logits, axis=1, keepdims=True)
    s = jnp.sum(jnp.exp(logits - m), axis=1, keepdims=True)
    w_ref[...] = 1.0 / s
    ncol = logits.shape[1]
    col = lax.broadcasted_iota(jnp.int32, logits.shape, 1)
    idx_ref[...] = jnp.min(jnp.where(logits >= m, col, ncol), axis=1, keepdims=True)


def _ffn_body(beff_ref, te_ref, nval_ref, xsrc, ws_src, w1a_ref, w1b_ref,
              b1a_ref, b1b_ref, w2a_ref, w2b_ref, b2_ref, out_ref,
              xbuf, wsbuf, xsem, wsem):
    t = pl.program_id(0)
    nt = pl.num_programs(0)

    def copies(tt, slot):
        base = pl.multiple_of(beff_ref[tt], 8)
        return (
            pltpu.make_async_copy(xsrc.at[pl.ds(base, BTE), :],
                                  xbuf.at[slot], xsem.at[slot]),
            pltpu.make_async_copy(ws_src.at[pl.ds(base, BTE), :],
                                  wsbuf.at[slot], wsem.at[slot]),
        )

    @pl.when(t == 0)
    def _():
        for c in copies(0, 0):
            c.start()

    @pl.when(t + 1 < nt)
    def _():
        for c in copies(jnp.minimum(t + 1, nt - 1), (t + 1) % 2):
            c.start()

    for c in copies(t, t % 2):
        c.wait()

    @pl.when(nval_ref[t] > 0)
    def _():
        xs = xbuf[t % 2]
        ha = jnp.maximum(
            jnp.dot(xs, w1a_ref[0], preferred_element_type=jnp.float32)
            + b1a_ref[0], 0.0)
        hb = jnp.maximum(
            jnp.dot(xs, w1b_ref[0], preferred_element_type=jnp.float32)
            + b1b_ref[0], 0.0)
        o = (jnp.dot(ha, w2a_ref[0], preferred_element_type=jnp.float32)
             + jnp.dot(hb, w2b_ref[0], preferred_element_type=jnp.float32))
        out_ref[...] = (o + b2_ref[0]) * wsbuf[t % 2]


def _route(idx, T, E, NTMAX):
    i32 = jnp.int32
    q = jnp.arange(T, dtype=i32)
    eq, order = lax.sort_key_val(idx.astype(i32), q)
    newseg = jnp.concatenate(
        [jnp.ones((1,), jnp.bool_), eq[1:] != eq[:-1]])
    segstart = lax.cummax(jnp.where(newseg, q, 0))
    off = q - segstart
    tq = jnp.cumsum(((off % BT) == 0).astype(i32)) - 1
    orig_q = q - (off % BT)
    beff_q = jnp.minimum(orig_q & ~7, T - BTE).astype(i32)
    src_row = (tq * BTE + (q - beff_q)).astype(i32)

    last_e = eq[T - 1]
    t_arr = jnp.arange(NTMAX, dtype=i32)
    active = (t_arr <= tq[T - 1]).astype(i32)
    pack = jnp.full((NTMAX,), last_e, i32).at[tq].set(beff_q * E + eq)
    te = pack % E
    beff = (pack // E) * active
    return order, te, active, beff, src_row


def _sc_gather(table, idx3, R, D):
    b_per_w = R // NW
    nch = b_per_w // CH
    mesh = plsc.VectorSubcoreMesh(core_axis_name="c", subcore_axis_name="s")

    @functools.partial(
        pl.kernel,
        out_type=jax.ShapeDtypeStruct((R, D), jnp.float32),
        mesh=mesh,
        scratch_types=[
            pltpu.VMEM((nch, CH), jnp.int32),
            pltpu.VMEM((CH, D), jnp.float32),
            pltpu.VMEM((CH, D), jnp.float32),
            pltpu.SemaphoreType.DMA,
            pltpu.SemaphoreType.DMA,
            pltpu.SemaphoreType.DMA,
            pltpu.SemaphoreType.DMA,
        ],
    )
    def k(table_hbm, idx_hbm, out_hbm, idx_v, buf_a, buf_b, ga, gb, oa, ob):
        wid = lax.axis_index("s") * NC + lax.axis_index("c")
        base = wid * b_per_w
        pltpu.sync_copy(idx_hbm.at[wid], idx_v)
        bufs = [(buf_a, ga, oa), (buf_b, gb, ob)]
        gh = pltpu.async_copy(table_hbm.at[idx_v.at[0]], buf_a, ga)
        oh = None
        for j in range(nch):
            buf, _, osem = bufs[j % 2]
            gh.wait()
            if oh is not None:
                oh.wait()
            if j + 1 < nch:
                nbuf, ngsem, _ = bufs[(j + 1) % 2]
                gh = pltpu.async_copy(table_hbm.at[idx_v.at[j + 1]], nbuf, ngsem)
            oh = pltpu.async_copy(buf, out_hbm.at[pl.ds(base + j * CH, CH)], osem)
        oh.wait()

    return k(table, idx3)


def _sc_scatter(src, sidx3, didx3, R, D):
    b_per_w = R // NW
    nch = b_per_w // CH
    mesh = plsc.VectorSubcoreMesh(core_axis_name="c", subcore_axis_name="s")

    @functools.partial(
        pl.kernel,
        out_type=jax.ShapeDtypeStruct((R, D), jnp.float32),
        mesh=mesh,
        scratch_types=[
            pltpu.VMEM((nch, CH), jnp.int32),
            pltpu.VMEM((nch, CH), jnp.int32),
            pltpu.VMEM((CH, D), jnp.float32),
            pltpu.VMEM((CH, D), jnp.float32),
            pltpu.SemaphoreType.DMA,
            pltpu.SemaphoreType.DMA,
            pltpu.SemaphoreType.DMA,
            pltpu.SemaphoreType.DMA,
        ],
    )
    def k(src_hbm, sidx_hbm, didx_hbm, out_hbm, sidx_v, didx_v,
          buf_a, buf_b, ga, gb, oa, ob):
        wid = lax.axis_index("s") * NC + lax.axis_index("c")
        pltpu.sync_copy(sidx_hbm.at[wid], sidx_v)
        pltpu.sync_copy(didx_hbm.at[wid], didx_v)
        bufs = [(buf_a, ga, oa), (buf_b, gb, ob)]
        gh = pltpu.async_copy(src_hbm.at[sidx_v.at[0]], buf_a, ga)
        oh = None
        for j in range(nch):
            buf, _, osem = bufs[j % 2]
            gh.wait()
            if oh is not None:
                oh.wait()
            if j + 1 < nch:
                nbuf, ngsem, _ = bufs[(j + 1) % 2]
                gh = pltpu.async_copy(src_hbm.at[sidx_v.at[j + 1]], nbuf, ngsem)
            oh = pltpu.async_copy(buf, out_hbm.at[didx_v.at[j]], osem)
        oh.wait()

    return k(src, sidx3, didx3)


def kernel(x, Wg, W1, b1, W2, b2):
    Bb, S, D = x.shape
    E, _, F = W1.shape
    T = Bb * S
    ntmax = T // BT + E - 1

    x_flat = x.reshape(T, D)

    idx2d, w2d = pl.pallas_call(
        _gate_body,
        out_shape=(
            jax.ShapeDtypeStruct((T, 1), jnp.int32),
            jax.ShapeDtypeStruct((T, 1), jnp.float32),
        ),
    )(x_flat, Wg)
    idx = idx2d[:, 0]

    order, te, nval, beff, src_row = _route(idx, T, E, ntmax)

    xs = _sc_gather(x_flat, order.reshape(NW, -1, CH), T, D)
    ws_sorted = w2d[order]

    grid_spec = pltpu.PrefetchScalarGridSpec(
        num_scalar_prefetch=3,
        grid=(ntmax,),
        in_specs=[
            pl.BlockSpec(memory_space=pl.ANY),
            pl.BlockSpec(memory_space=pl.ANY),
            pl.BlockSpec((1, D, F // 2), lambda i, be, te_r, nv: (te_r[i], 0, 0)),
            pl.BlockSpec((1, D, F // 2), lambda i, be, te_r, nv: (te_r[i], 0, 1)),
            pl.BlockSpec((1, 1, F // 2), lambda i, be, te_r, nv: (te_r[i], 0, 0)),
            pl.BlockSpec((1, 1, F // 2), lambda i, be, te_r, nv: (te_r[i], 0, 1)),
            pl.BlockSpec((1, F // 2, D), lambda i, be, te_r, nv: (te_r[i], 0, 0)),
            pl.BlockSpec((1, F // 2, D), lambda i, be, te_r, nv: (te_r[i], 1, 0)),
            pl.BlockSpec((1, 1, D), lambda i, be, te_r, nv: (te_r[i], 0, 0)),
        ],
        out_specs=pl.BlockSpec((BTE, D), lambda i, be, te_r, nv: (i, 0)),
        scratch_shapes=[
            pltpu.VMEM((2, BTE, D), jnp.float32),
            pltpu.VMEM((2, BTE, 1), jnp.float32),
            pltpu.SemaphoreType.DMA((2,)),
            pltpu.SemaphoreType.DMA((2,)),
        ],
    )
    ffn_out = pl.pallas_call(
        _ffn_body,
        grid_spec=grid_spec,
        out_shape=jax.ShapeDtypeStruct((ntmax * BTE, D), jnp.float32),
        compiler_params=pltpu.CompilerParams(
            dimension_semantics=("arbitrary",),
        ),
    )(beff, te, nval, xs, ws_sorted, W1, W1, b1.reshape(E, 1, F),
      b1.reshape(E, 1, F), W2, W2, b2.reshape(E, 1, D))

    out = _sc_scatter(ffn_out, src_row.reshape(NW, -1, CH),
                      order.reshape(NW, -1, CH), T, D)
    return out.reshape(Bb, S, D)


# speedup vs baseline: 1.0743x; 1.0743x over previous
import functools

import jax
import jax.numpy as jnp
from jax import lax
from jax.experimental import pallas as pl
from jax.experimental.pallas import tpu as pltpu
from jax.experimental.pallas import tpu_sc as plsc

BT = 128
BTE = BT + 8
NC = 2
NS = 16
NW = NC * NS
CH = 64


def _gate_body(x_ref, wg_ref, idx_ref, w_ref):
    logits = jnp.dot(x_ref[...], wg_ref[...], preferred_element_type=jnp.float32)
    m = jnp.max(logits, axis=1, keepdims=True)
    s = jnp.sum(jnp.exp(logits - m), axis=1, keepdims=True)
    w_ref[...] = 1.0 / s
    ncol = logits.shape[1]
    col = lax.broadcasted_iota(jnp.int32, logits.shape, 1)
    idx_ref[...] = jnp.min(jnp.where(logits >= m, col, ncol), axis=1, keepdims=True)


def _ffn_body(beff_ref, te_ref, xsrc, ws_src, w1a_ref, w1b_ref,
              b1a_ref, b1b_ref, w2a_ref, w2b_ref, b2_ref, out_ref,
              xbuf, wsbuf, xsem, wsem):
    t = pl.program_id(0)
    nt = pl.num_programs(0)

    def copies(tt, slot):
        base = pl.multiple_of(beff_ref[tt], 8)
        return (
            pltpu.make_async_copy(xsrc.at[pl.ds(base, BTE), :],
                                  xbuf.at[slot], xsem.at[slot]),
            pltpu.make_async_copy(ws_src.at[pl.ds(base, BTE), :],
                                  wsbuf.at[slot], wsem.at[slot]),
        )

    @pl.when(t == 0)
    def _():
        for c in copies(0, 0):
            c.start()

    @pl.when(t + 1 < nt)
    def _():
        for c in copies(jnp.minimum(t + 1, nt - 1), (t + 1) % 2):
            c.start()

    for c in copies(t, t % 2):
        c.wait()

    xs = xbuf[t % 2]
    ha = jnp.maximum(
        jnp.dot(xs, w1a_ref[0], preferred_element_type=jnp.float32)
        + b1a_ref[0], 0.0)
    hb = jnp.maximum(
        jnp.dot(xs, w1b_ref[0], preferred_element_type=jnp.float32)
        + b1b_ref[0], 0.0)
    o = (jnp.dot(ha, w2a_ref[0], preferred_element_type=jnp.float32)
         + jnp.dot(hb, w2b_ref[0], preferred_element_type=jnp.float32))
    out_ref[...] = (o + b2_ref[0]) * wsbuf[t % 2]


def _route(idx, T, E, NTMAX):
    i32 = jnp.int32
    q = jnp.arange(T, dtype=i32)
    eq, order = lax.sort_key_val(idx.astype(i32), q)
    newseg = jnp.concatenate(
        [jnp.ones((1,), jnp.bool_), eq[1:] != eq[:-1]])
    segstart = lax.cummax(jnp.where(newseg, q, 0))
    off = q - segstart
    tq = jnp.cumsum(((off % BT) == 0).astype(i32)) - 1
    orig_q = q - (off % BT)
    beff_q = jnp.minimum(orig_q & ~7, T - BTE).astype(i32)
    src_row = (tq * BTE + (q - beff_q)).astype(i32)

    last_e = eq[T - 1]
    n_tiles = tq[T - 1] + 1
    pack = jnp.full((NTMAX,), last_e, i32).at[tq].set(beff_q * E + eq)
    te = pack % E
    beff = pack // E
    return order, te, n_tiles, beff, src_row


def _sc_gather(table, idx3, R, D):
    b_per_w = R // NW
    nch = b_per_w // CH
    mesh = plsc.VectorSubcoreMesh(core_axis_name="c", subcore_axis_name="s")

    @functools.partial(
        pl.kernel,
        out_type=jax.ShapeDtypeStruct((R, D), jnp.float32),
        mesh=mesh,
        scratch_types=[
            pltpu.VMEM((nch, CH), jnp.int32),
            pltpu.VMEM((CH, D), jnp.float32),
            pltpu.VMEM((CH, D), jnp.float32),
            pltpu.SemaphoreType.DMA,
            pltpu.SemaphoreType.DMA,
            pltpu.SemaphoreType.DMA,
            pltpu.SemaphoreType.DMA,
        ],
    )
    def k(table_hbm, idx_hbm, out_hbm, idx_v, buf_a, buf_b, ga, gb, oa, ob):
        wid = lax.axis_index("s") * NC + lax.axis_index("c")
        base = wid * b_per_w
        pltpu.sync_copy(idx_hbm.at[wid], idx_v)
        bufs = [(buf_a, ga, oa), (buf_b, gb, ob)]
        gh = pltpu.async_copy(table_hbm.at[idx_v.at[0]], buf_a, ga)
        oh = None
        for j in range(nch):
            buf, _, osem = bufs[j % 2]
            gh.wait()
            if oh is not None:
                oh.wait()
            if j + 1 < nch:
                nbuf, ngsem, _ = bufs[(j + 1) % 2]
                gh = pltpu.async_copy(table_hbm.at[idx_v.at[j + 1]], nbuf, ngsem)
            oh = pltpu.async_copy(buf, out_hbm.at[pl.ds(base + j * CH, CH)], osem)
        oh.wait()

    return k(table, idx3)


def _sc_scatter(src, sidx3, didx3, R, D):
    b_per_w = R // NW
    nch = b_per_w // CH
    mesh = plsc.VectorSubcoreMesh(core_axis_name="c", subcore_axis_name="s")

    @functools.partial(
        pl.kernel,
        out_type=jax.ShapeDtypeStruct((R, D), jnp.float32),
        mesh=mesh,
        scratch_types=[
            pltpu.VMEM((nch, CH), jnp.int32),
            pltpu.VMEM((nch, CH), jnp.int32),
            pltpu.VMEM((CH, D), jnp.float32),
            pltpu.VMEM((CH, D), jnp.float32),
            pltpu.SemaphoreType.DMA,
            pltpu.SemaphoreType.DMA,
            pltpu.SemaphoreType.DMA,
            pltpu.SemaphoreType.DMA,
        ],
    )
    def k(src_hbm, sidx_hbm, didx_hbm, out_hbm, sidx_v, didx_v,
          buf_a, buf_b, ga, gb, oa, ob):
        wid = lax.axis_index("s") * NC + lax.axis_index("c")
        pltpu.sync_copy(sidx_hbm.at[wid], sidx_v)
        pltpu.sync_copy(didx_hbm.at[wid], didx_v)
        bufs = [(buf_a, ga, oa), (buf_b, gb, ob)]
        gh = pltpu.async_copy(src_hbm.at[sidx_v.at[0]], buf_a, ga)
        oh = None
        for j in range(nch):
            buf, _, osem = bufs[j % 2]
            gh.wait()
            if oh is not None:
                oh.wait()
            if j + 1 < nch:
                nbuf, ngsem, _ = bufs[(j + 1) % 2]
                gh = pltpu.async_copy(src_hbm.at[sidx_v.at[j + 1]], nbuf, ngsem)
            oh = pltpu.async_copy(buf, out_hbm.at[didx_v.at[j]], osem)
        oh.wait()

    return k(src, sidx3, didx3)


def kernel(x, Wg, W1, b1, W2, b2):
    Bb, S, D = x.shape
    E, _, F = W1.shape
    T = Bb * S
    ntmax = T // BT + E - 1

    x_flat = x.reshape(T, D)

    idx2d, w2d = pl.pallas_call(
        _gate_body,
        out_shape=(
            jax.ShapeDtypeStruct((T, 1), jnp.int32),
            jax.ShapeDtypeStruct((T, 1), jnp.float32),
        ),
    )(x_flat, Wg)
    idx = idx2d[:, 0]

    order, te, n_tiles, beff, src_row = _route(idx, T, E, ntmax)

    xs = _sc_gather(x_flat, order.reshape(NW, -1, CH), T, D)
    ws_sorted = w2d[order]

    grid_spec = pltpu.PrefetchScalarGridSpec(
        num_scalar_prefetch=2,
        grid=(n_tiles,),
        in_specs=[
            pl.BlockSpec(memory_space=pl.ANY),
            pl.BlockSpec(memory_space=pl.ANY),
            pl.BlockSpec((1, D, F // 2), lambda i, be, te_r: (te_r[i], 0, 0)),
            pl.BlockSpec((1, D, F // 2), lambda i, be, te_r: (te_r[i], 0, 1)),
            pl.BlockSpec((1, 1, F // 2), lambda i, be, te_r: (te_r[i], 0, 0)),
            pl.BlockSpec((1, 1, F // 2), lambda i, be, te_r: (te_r[i], 0, 1)),
            pl.BlockSpec((1, F // 2, D), lambda i, be, te_r: (te_r[i], 0, 0)),
            pl.BlockSpec((1, F // 2, D), lambda i, be, te_r: (te_r[i], 1, 0)),
            pl.BlockSpec((1, 1, D), lambda i, be, te_r: (te_r[i], 0, 0)),
        ],
        out_specs=pl.BlockSpec((BTE, D), lambda i, be, te_r: (i, 0)),
        scratch_shapes=[
            pltpu.VMEM((2, BTE, D), jnp.float32),
            pltpu.VMEM((2, BTE, 1), jnp.float32),
            pltpu.SemaphoreType.DMA((2,)),
            pltpu.SemaphoreType.DMA((2,)),
        ],
    )
    ffn_out = pl.pallas_call(
        _ffn_body,
        grid_spec=grid_spec,
        out_shape=jax.ShapeDtypeStruct((ntmax * BTE, D), jnp.float32),
        compiler_params=pltpu.CompilerParams(
            dimension_semantics=("arbitrary",),
        ),
    )(beff, te, xs, ws_sorted, W1, W1, b1.reshape(E, 1, F),
      b1.reshape(E, 1, F), W2, W2, b2.reshape(E, 1, D))

    out = _sc_scatter(ffn_out, src_row.reshape(NW, -1, CH),
                      order.reshape(NW, -1, CH), T, D)
    return out.reshape(Bb, S, D)
